# baseline (speedup 1.0000x reference)
import functools

import jax
import jax.numpy as jnp
from jax import lax
from jax.experimental import pallas as pl
from jax.experimental.pallas import tpu as pltpu
from jax.experimental.pallas import tpu_sc as plsc

D_MODEL = 768
HALF = D_MODEL // 2
H = 256
W = 256
NW = 32
CPW = HALF // NW
REP_ROWS = 32
NREP = H // REP_ROWS
STRIPE_W = 128
NJ = W // 16
NSTEP = CPW // 2


def _body(tab, out, rows1_v, rows2_v, rep0, rep1, st0, st1,
          sem_r0, sem_r1, sem_s0, sem_s1):
    wid = lax.axis_index("s") * 2 + lax.axis_index("c")
    reps = (rep0, rep1)
    stripes = (st0, st1)
    sems_r = (sem_r0, sem_r1)
    sems_s = (sem_s0, sem_s1)

    pltpu.sync_copy(tab.at[pl.ds(wid * CPW * W, CPW * W)], rows1_v)
    pltpu.sync_copy(tab.at[pl.ds((HALF + wid * CPW) * W, CPW * W)], rows2_v)

    def fill_fire(k, b):
        c1 = wid * CPW + k
        c2 = HALF + wid * CPW + k
        rep_v, stripe_v = reps[b], stripes[b]

        chunks = [rows1_v[pl.ds(k * W + j * 16, 16)] for j in range(NJ)]

        def fill_rep(r, carry):
            for j in range(NJ):
                rep_v[r, pl.ds(j * 16, 16)] = chunks[j]
            return carry

        lax.fori_loop(0, REP_ROWS, fill_rep, 0)

        def fire_rep(d, carry):
            pltpu.async_copy(
                rep_v, out.at[c1, pl.ds(d * REP_ROWS, REP_ROWS)], sems_r[b])
            return carry

        lax.fori_loop(0, NREP, fire_rep, 0)

        def fill_chunk(cb, carry):
            chunk = rows2_v[pl.ds(k * W + cb * 16, 16)]
            for lane in range(16):
                val = jnp.full((16,), chunk[lane])
                r = cb * 16 + lane
                for j in range(STRIPE_W // 16):
                    stripe_v[r, pl.ds(j * 16, 16)] = val
            return carry

        lax.fori_loop(0, H // 16, fill_chunk, 0)
        pltpu.async_copy(stripe_v, out.at[c2, :, pl.ds(0, STRIPE_W)],
                         sems_s[b])
        pltpu.async_copy(stripe_v, out.at[c2, :, pl.ds(STRIPE_W, STRIPE_W)],
                         sems_s[b])

    def drain(k, b):
        c1 = wid * CPW + k
        c2 = HALF + wid * CPW + k

        def drain_rep(d, carry):
            pltpu.make_async_copy(
                reps[b], out.at[c1, pl.ds(d * REP_ROWS, REP_ROWS)], sems_r[b]
            ).wait()
            return carry

        lax.fori_loop(0, NREP, drain_rep, 0)
        pltpu.make_async_copy(
            stripes[b], out.at[c2, :, pl.ds(0, STRIPE_W)], sems_s[b]).wait()
        pltpu.make_async_copy(
            stripes[b], out.at[c2, :, pl.ds(STRIPE_W, STRIPE_W)],
            sems_s[b]).wait()

    fill_fire(0, 0)
    fill_fire(1, 1)

    def step(t, carry):
        for b in range(2):
            drain(2 * t - 2 + b, b)
            fill_fire(2 * t + b, b)
        return carry

    lax.fori_loop(1, NSTEP, step, 0)
    drain(2 * NSTEP - 2, 0)
    drain(2 * NSTEP - 1, 1)


@jax.jit
def _pos_encode(tab):
    mesh = plsc.VectorSubcoreMesh(core_axis_name="c", subcore_axis_name="s")
    fn = functools.partial(
        pl.kernel,
        mesh=mesh,
        out_type=jax.ShapeDtypeStruct((D_MODEL, H, W), jnp.float32),
        scratch_types=[
            pltpu.VMEM((CPW * W,), jnp.float32),
            pltpu.VMEM((CPW * W,), jnp.float32),
            pltpu.VMEM((REP_ROWS, W), jnp.float32),
            pltpu.VMEM((REP_ROWS, W), jnp.float32),
            pltpu.VMEM((H, STRIPE_W), jnp.float32),
            pltpu.VMEM((H, STRIPE_W), jnp.float32),
            pltpu.SemaphoreType.DMA,
            pltpu.SemaphoreType.DMA,
            pltpu.SemaphoreType.DMA,
            pltpu.SemaphoreType.DMA,
        ],
    )(_body)
    return fn(tab)


def kernel(x, row_w, col_w):
    h = min(x.shape[-2], row_w.shape[0])
    w = min(x.shape[-1], col_w.shape[0])
    assert (h, w) == (H, W) and row_w.shape[1] == HALF
    tab = jnp.concatenate([col_w[:w].T, row_w[:h].T], axis=0).reshape(-1)
    out = _pos_encode(tab)
    return out[None, ...]


# speedup vs baseline: 1.0639x; 1.0639x over previous
import functools

import jax
import jax.numpy as jnp
from jax import lax
from jax.experimental import pallas as pl
from jax.experimental.pallas import tpu as pltpu
from jax.experimental.pallas import tpu_sc as plsc

D_MODEL = 768
HALF = D_MODEL // 2
H = 256
W = 256
NW = 32
CPW = HALF // NW
REP_ROWS = 8
NREP = H // REP_ROWS
STRIPE_W = 128
NJ = W // 16


def _body(tab, out, rows1_v, rows2_v, rep_v, stripe_v, sem_r, sem_h0, sem_h1):
    wid = lax.axis_index("s") * 2 + lax.axis_index("c")

    pltpu.sync_copy(tab.at[pl.ds(wid * CPW * W, CPW * W)], rows1_v)
    pltpu.sync_copy(tab.at[pl.ds((HALF + wid * CPW) * W, CPW * W)], rows2_v)

    def plane_pair(k, carry):
        c1 = wid * CPW + k
        c2 = HALF + wid * CPW + k

        chunks = [rows1_v[pl.ds(k * W + j * 16, 16)] for j in range(NJ)]

        def fill_rep(r, carry2):
            for j in range(NJ):
                rep_v[r, pl.ds(j * 16, 16)] = chunks[j]
            return carry2

        lax.fori_loop(0, REP_ROWS, fill_rep, 0)

        @pl.when(k > 0)
        def _():
            c2p = c2 - 1
            pltpu.make_async_copy(
                stripe_v, out.at[c2p, :, pl.ds(0, STRIPE_W)], sem_h0).wait()
            pltpu.make_async_copy(
                stripe_v, out.at[c2p, :, pl.ds(STRIPE_W, STRIPE_W)],
                sem_h1).wait()

        def fire_rep(d, carry2):
            pltpu.async_copy(
                rep_v, out.at[c1, pl.ds(d * REP_ROWS, REP_ROWS)], sem_r)
            return carry2

        lax.fori_loop(0, NREP, fire_rep, 0)

        def fill_chunk(cb, carry2):
            chunk = rows2_v[pl.ds(k * W + cb * 16, 16)]
            for lane in range(16):
                val = jnp.full((16,), chunk[lane])
                r = cb * 16 + lane
                for j in range(STRIPE_W // 16):
                    stripe_v[r, pl.ds(j * 16, 16)] = val
            return carry2

        lax.fori_loop(0, H // 16, fill_chunk, 0)
        pltpu.async_copy(stripe_v, out.at[c2, :, pl.ds(0, STRIPE_W)], sem_h0)
        pltpu.async_copy(
            stripe_v, out.at[c2, :, pl.ds(STRIPE_W, STRIPE_W)], sem_h1)

        def drain_rep(d, carry2):
            pltpu.make_async_copy(
                rep_v, out.at[c1, pl.ds(d * REP_ROWS, REP_ROWS)], sem_r
            ).wait()
            return carry2

        lax.fori_loop(0, NREP, drain_rep, 0)
        return carry

    lax.fori_loop(0, CPW, plane_pair, 0)
    c2last = HALF + wid * CPW + CPW - 1
    pltpu.make_async_copy(
        stripe_v, out.at[c2last, :, pl.ds(0, STRIPE_W)], sem_h0).wait()
    pltpu.make_async_copy(
        stripe_v, out.at[c2last, :, pl.ds(STRIPE_W, STRIPE_W)], sem_h1).wait()


@jax.jit
def _pos_encode(tab):
    mesh = plsc.VectorSubcoreMesh(core_axis_name="c", subcore_axis_name="s")
    fn = functools.partial(
        pl.kernel,
        mesh=mesh,
        out_type=jax.ShapeDtypeStruct((D_MODEL, H, W), jnp.float32),
        scratch_types=[
            pltpu.VMEM((CPW * W,), jnp.float32),
            pltpu.VMEM((CPW * W,), jnp.float32),
            pltpu.VMEM((REP_ROWS, W), jnp.float32),
            pltpu.VMEM((H, STRIPE_W), jnp.float32),
            pltpu.SemaphoreType.DMA,
            pltpu.SemaphoreType.DMA,
            pltpu.SemaphoreType.DMA,
        ],
    )(_body)
    return fn(tab)


def kernel(x, row_w, col_w):
    h = min(x.shape[-2], row_w.shape[0])
    w = min(x.shape[-1], col_w.shape[0])
    assert (h, w) == (H, W) and row_w.shape[1] == HALF
    tab = jnp.concatenate([col_w[:w].T, row_w[:h].T], axis=0).reshape(-1)
    out = _pos_encode(tab)
    return out[None, ...]
